# Initial kernel scaffold; baseline (speedup 1.0000x reference)
import jax
import jax.numpy as jnp
from jax.experimental import pallas as pl


def kernel(x, edge_index, Wl0, bl0, Wr0, g0, be0, Wl1, bl1, Wr1, g1, be1, Wl2, bl2, Wr2, g2, be2, W1, b1, W2, b2):
    raise NotImplementedError("write your pallas kernel here")


# speedup vs baseline: 5.9348x; 5.9348x over previous
import functools

import jax
import jax.numpy as jnp
from jax import lax
from jax.experimental import pallas as pl
from jax.experimental.pallas import tpu as pltpu
from jax.experimental.pallas import tpu_sc as plsc

N = 10000
E = 320000
D = 128
NC = 2
NS = 16
NW = NC * NS
EPW = E // NW
C = 80
NCH = EPW // C
NP = 10240
RPT = NP // NS
R = 1000
GRID = N // R


def _sc_degree(dst3, z128, ones128):
  mesh = plsc.VectorSubcoreMesh(core_axis_name="c", subcore_axis_name="s")
  scratch = [
      pltpu.VMEM((NCH, C), jnp.int32),
      pltpu.VMEM((C,), jnp.int32),
      pltpu.VMEM((C, D), jnp.float32),
      pltpu.VMEM_SHARED((NP, D), jnp.float32),
  ]

  @functools.partial(
      pl.kernel, mesh=mesh,
      out_type=jax.ShapeDtypeStruct((NC, NP, D), jnp.float32),
      scratch_types=scratch)
  def k(dst_hbm, z128_hbm, ones_hbm, deg_hbm, dst_v, dst_cv, ones_v, dacc):
    c = lax.axis_index("c")
    s = lax.axis_index("s")
    wid = c * NS + s
    pltpu.sync_copy(dst_hbm.at[wid], dst_v)
    pltpu.sync_copy(ones_hbm, ones_v)
    pltpu.sync_copy(z128_hbm, dacc.at[pl.ds(s * RPT, RPT)])
    plsc.subcore_barrier()

    @pl.loop(0, NCH)
    def _(j):
      @pl.loop(0, C, step=16)
      def _(q):
        dst_cv[pl.ds(q, 16)] = dst_v[j, pl.ds(q, 16)]
      pltpu.sync_copy(ones_v, dacc.at[dst_cv], add=True)

    plsc.subcore_barrier()
    pltpu.sync_copy(dacc.at[pl.ds(s * RPT, RPT)],
                    deg_hbm.at[c].at[pl.ds(s * RPT, RPT)])

  return k(dst3, z128, ones128)


def _sc_aggregate(h, src3, dst3, z128):
  mesh = plsc.VectorSubcoreMesh(core_axis_name="c", subcore_axis_name="s")
  scratch = [
      pltpu.VMEM((NCH, C), jnp.int32),
      pltpu.VMEM((NCH, C), jnp.int32),
      pltpu.VMEM((C,), jnp.int32),
      pltpu.VMEM((C, D), jnp.float32),
      pltpu.VMEM_SHARED((NP, D), jnp.float32),
      pltpu.SemaphoreType.DMA,
  ]

  @functools.partial(
      pl.kernel, mesh=mesh,
      out_type=jax.ShapeDtypeStruct((NC, NP, D), jnp.float32),
      scratch_types=scratch)
  def k(h_hbm, src_hbm, dst_hbm, z128_hbm, agg_hbm,
        src_v, dst_v, dst_cv, rows_v, acc, sem):
    c = lax.axis_index("c")
    s = lax.axis_index("s")
    wid = c * NS + s
    pltpu.sync_copy(src_hbm.at[wid], src_v)
    pltpu.sync_copy(dst_hbm.at[wid], dst_v)
    pltpu.sync_copy(z128_hbm, acc.at[pl.ds(s * RPT, RPT)])
    plsc.subcore_barrier()

    @pl.loop(0, NCH)
    def _(j):
      @pl.loop(0, C, step=16)
      def _(q):
        dst_cv[pl.ds(q, 16)] = dst_v[j, pl.ds(q, 16)]
      pltpu.async_copy(h_hbm.at[src_v.at[j]], rows_v, sem).wait()
      pltpu.sync_copy(rows_v, acc.at[dst_cv], add=True)

    plsc.subcore_barrier()
    pltpu.sync_copy(acc.at[pl.ds(s * RPT, RPT)],
                    agg_hbm.at[c].at[pl.ds(s * RPT, RPT)])

  return k(h, src3, dst3, z128)


def _tc_linear_stats(agg2, deg2, h, wlt, bl, wrt):
  def body(agg_ref, deg_ref, h_ref, wl_ref, bl_ref, wr_ref,
           z_ref, st_ref, acc_ref):
    i = pl.program_id(0)
    a = agg_ref[0] + agg_ref[1]
    dg = jnp.maximum(deg_ref[0, :, 0:1] + deg_ref[1, :, 0:1], 1.0)
    mean = a / dg
    z = (jnp.dot(mean, wl_ref[...], preferred_element_type=jnp.float32)
         + bl_ref[...]
         + jnp.dot(h_ref[...], wr_ref[...], preferred_element_type=jnp.float32))
    z_ref[...] = z

    @pl.when(i == 0)
    def _():
      acc_ref[...] = jnp.zeros_like(acc_ref)

    acc_ref[0:1, :] += jnp.sum(z, axis=0, keepdims=True)
    acc_ref[1:2, :] += jnp.sum(z * z, axis=0, keepdims=True)
    st_ref[...] = acc_ref[...]

  return pl.pallas_call(
      body,
      grid=(GRID,),
      in_specs=[
          pl.BlockSpec((NC, R, D), lambda i: (0, i, 0)),
          pl.BlockSpec((NC, R, D), lambda i: (0, i, 0)),
          pl.BlockSpec((R, D), lambda i: (i, 0)),
          pl.BlockSpec((D, D), lambda i: (0, 0)),
          pl.BlockSpec((1, D), lambda i: (0, 0)),
          pl.BlockSpec((D, D), lambda i: (0, 0)),
      ],
      out_specs=[
          pl.BlockSpec((R, D), lambda i: (i, 0)),
          pl.BlockSpec((2, D), lambda i: (0, 0)),
      ],
      out_shape=[
          jax.ShapeDtypeStruct((N, D), jnp.float32),
          jax.ShapeDtypeStruct((2, D), jnp.float32),
      ],
      scratch_shapes=[pltpu.VMEM((2, D), jnp.float32)],
  )(agg2, deg2, h, wlt, bl, wrt)


def _tc_bn_relu(z, st, g, be):
  def body(z_ref, st_ref, g_ref, be_ref, o_ref):
    mu = st_ref[0:1, :] * (1.0 / N)
    var = st_ref[1:2, :] * (1.0 / N) - mu * mu
    inv = lax.rsqrt(var + 1e-5)
    o_ref[...] = jnp.maximum(
        g_ref[...] * (z_ref[...] - mu) * inv + be_ref[...], 0.0)

  return pl.pallas_call(
      body,
      grid=(GRID,),
      in_specs=[
          pl.BlockSpec((R, D), lambda i: (i, 0)),
          pl.BlockSpec((2, D), lambda i: (0, 0)),
          pl.BlockSpec((1, D), lambda i: (0, 0)),
          pl.BlockSpec((1, D), lambda i: (0, 0)),
      ],
      out_specs=pl.BlockSpec((R, D), lambda i: (i, 0)),
      out_shape=jax.ShapeDtypeStruct((N, D), jnp.float32),
  )(z, st, g, be)


def _tc_classifier(h, w1t, b1, w2t, b2):
  def body(h_ref, w1_ref, b1_ref, w2_ref, b2_ref, p_ref):
    t = jnp.maximum(
        jnp.dot(h_ref[...], w1_ref[...], preferred_element_type=jnp.float32)
        + b1_ref[...], 0.0)
    lg = (jnp.dot(t, w2_ref[...], preferred_element_type=jnp.float32)
          + b2_ref[...])
    m = jnp.max(lg, axis=1, keepdims=True)
    e = jnp.exp(lg - m)
    p_ref[...] = e / jnp.sum(e, axis=1, keepdims=True)

  return pl.pallas_call(
      body,
      grid=(GRID,),
      in_specs=[
          pl.BlockSpec((R, D), lambda i: (i, 0)),
          pl.BlockSpec((D, 64), lambda i: (0, 0)),
          pl.BlockSpec((1, 64), lambda i: (0, 0)),
          pl.BlockSpec((64, 10), lambda i: (0, 0)),
          pl.BlockSpec((1, 10), lambda i: (0, 0)),
      ],
      out_specs=pl.BlockSpec((R, 10), lambda i: (i, 0)),
      out_shape=jax.ShapeDtypeStruct((N, 10), jnp.float32),
  )(h, w1t, b1, w2t, b2)


def kernel(x, edge_index, Wl0, bl0, Wr0, g0, be0, Wl1, bl1, Wr1, g1, be1,
           Wl2, bl2, Wr2, g2, be2, W1, b1, W2, b2):
  src3 = edge_index[0].reshape(NW, NCH, C)
  dst3 = edge_index[1].reshape(NW, NCH, C)
  z128 = jnp.zeros((RPT, D), jnp.float32)
  ones128 = jnp.ones((C, D), jnp.float32)

  deg2 = _sc_degree(dst3, z128, ones128)
  h = x
  for (Wl, bl, Wr, g, be) in ((Wl0, bl0, Wr0, g0, be0),
                              (Wl1, bl1, Wr1, g1, be1),
                              (Wl2, bl2, Wr2, g2, be2)):
    agg2 = _sc_aggregate(h, src3, dst3, z128)
    z, st = _tc_linear_stats(agg2, deg2, h, Wl.T, bl.reshape(1, D),
                             Wr.T)
    h = _tc_bn_relu(z, st, g.reshape(1, D), be.reshape(1, D))

  p = _tc_classifier(h, W1.T, b1.reshape(1, 64), W2.T, b2.reshape(1, 10))
  return (h, p)
